# baseline (speedup 1.0000x reference)
import functools

import jax
import jax.numpy as jnp
from jax import lax
from jax.experimental import pallas as pl
from jax.experimental.pallas import tpu as pltpu
from jax.experimental.pallas import tpu_sc as plsc

N_FINE = 100000
N_COARSE = 25000
VAL_DIM = 128
FE = 9
NF = 128
KDIM = FE * VAL_DIM

NC_SC = 2
NS_SC = 16
NW = NC_SC * NS_SC

M_BLK = 512
M_PAD = 25088
CHUNK = 128
SLAB_BLOCKS = (25, 24)


def _make_sc_gather(tot_rows):
    n_chunks = tot_rows // CHUNK
    base_iters = n_chunks // NW
    rem = n_chunks - base_iters * NW
    max_iters = base_iters + (1 if rem else 0)
    stage_rows = -(-(max_iters + 8) // 8) * 8
    chunks_pad = n_chunks + stage_rows
    mesh = plsc.VectorSubcoreMesh(core_axis_name="c", subcore_axis_name="s")

    @functools.partial(
        pl.kernel,
        mesh=mesh,
        out_type=jax.ShapeDtypeStruct((tot_rows, VAL_DIM), jnp.float32),
        scratch_types=[
            pltpu.VMEM((stage_rows, CHUNK), jnp.int32),
            pltpu.VMEM((2, CHUNK, VAL_DIM), jnp.float32),
            pltpu.SemaphoreType.DMA((2,)),
            pltpu.SemaphoreType.DMA((2,)),
        ],
    )
    def gather_kernel(fine_hbm, idx_hbm, out_hbm, idx_v, rows_v, gsem, wsem):
        wid = lax.axis_index("s") * NC_SC + lax.axis_index("c")
        first = wid * base_iters + jnp.minimum(wid, rem)
        n = base_iters + (wid < rem).astype(jnp.int32)

        aligned = pl.multiple_of((first // 8) * 8, 8)
        off = first - aligned
        pltpu.sync_copy(idx_hbm.at[pl.ds(aligned, stage_rows)], idx_v)

        def start_gather(j, slot):
            pltpu.async_copy(fine_hbm.at[idx_v.at[j + off]], rows_v.at[slot],
                             gsem.at[slot])

        def wait_gather(slot):
            pltpu.make_async_copy(fine_hbm.at[idx_v.at[0]], rows_v.at[slot],
                                  gsem.at[slot]).wait()

        def start_write(j, slot):
            dst = pl.multiple_of((first + j) * CHUNK, CHUNK)
            pltpu.async_copy(rows_v.at[slot], out_hbm.at[pl.ds(dst, CHUNK)],
                             wsem.at[slot])

        def wait_write(slot):
            pltpu.make_async_copy(rows_v.at[slot],
                                  out_hbm.at[pl.ds(0, CHUNK)],
                                  wsem.at[slot]).wait()

        start_gather(0, 0)

        def body(j, carry):
            slot = lax.rem(j, 2)
            nslot = 1 - slot

            @pl.when(j + 1 < n)
            def _():
                @pl.when(j >= 1)
                def _():
                    wait_write(nslot)
                start_gather(j + 1, nslot)

            wait_gather(slot)
            start_write(j, slot)
            return carry

        lax.fori_loop(0, n, body, 0)

        @pl.when(n >= 2)
        def _():
            wait_write(lax.rem(n, 2))

        wait_write(lax.rem(n - 1, 2))

    return gather_kernel, chunks_pad


def _mm_body(a_ref, w_ref, o_ref):
    a = jnp.concatenate([a_ref[i] for i in range(FE)], axis=1)
    o_ref[...] = jnp.dot(a, w_ref[...], preferred_element_type=jnp.float32)


def _tc_matmul(a3, w, m_pad, m_out):
    grid = (m_pad // M_BLK,)
    return pl.pallas_call(
        _mm_body,
        grid=grid,
        in_specs=[
            pl.BlockSpec((FE, M_BLK, VAL_DIM), lambda m: (0, m, 0)),
            pl.BlockSpec((KDIM, NF), lambda m: (0, 0)),
        ],
        out_specs=pl.BlockSpec((M_BLK, NF), lambda m: (m, 0)),
        out_shape=jax.ShapeDtypeStruct((m_out, NF), jnp.float32),
    )(a3, w)


def kernel(lattice_fine_values, coarse_neighbor_indices, weight):
    idx = coarse_neighbor_indices.astype(jnp.int32)
    idx_t = jnp.pad(idx.T, ((0, 0), (0, M_PAD - N_COARSE)))

    outs = []
    col = 0
    for s, nblk in enumerate(SLAB_BLOCKS):
        s_pad = nblk * M_BLK
        tot_rows = FE * s_pad
        gather_fn, chunks_pad = _make_sc_gather(tot_rows)
        idx_s = idx_t[:, col:col + s_pad].reshape(-1)
        idx2d = jnp.pad(idx_s, (0, chunks_pad * CHUNK - tot_rows))
        idx2d = idx2d.reshape(chunks_pad, CHUNK)
        gathered = gather_fn(lattice_fine_values, idx2d)
        a3 = gathered.reshape(FE, s_pad, VAL_DIM)
        m_out = min(N_COARSE - col, s_pad)
        outs.append(_tc_matmul(a3, weight, s_pad, m_out))
        col += s_pad
    return jnp.concatenate(outs, axis=0)
